# baseline (speedup 1.0000x reference)
import functools

import numpy as np

import jax
import jax.numpy as jnp
from jax import lax
from jax.experimental import pallas as pl
from jax.experimental.pallas import tpu as pltpu
from jax.experimental.pallas import tpu_sc as plsc

B = 4
L = 4096
D = 1024
SCALE = float(D) ** 0.5

NC = 2
NS = 16
NW = NC * NS
TPW = (B * L) // NW
C = 32
NSTEP = TPW // C


def _sc_gather(table, xf):
    mesh = plsc.VectorSubcoreMesh(
        core_axis_name="c", subcore_axis_name="s", num_cores=NC, num_subcores=NS
    )

    @functools.partial(
        pl.kernel,
        out_type=jax.ShapeDtypeStruct((B * L, D), jnp.float32),
        mesh=mesh,
        scratch_types=(
            [pltpu.VMEM((TPW,), jnp.int32)]
            + [pltpu.VMEM((C, D), jnp.float32) for _ in range(3)]
            + [pltpu.SemaphoreType.DMA for _ in range(6)]
        ),
    )
    def body(table_hbm, xf_hbm, out_hbm, idxbuf, r0, r1, r2,
             g0, g1, g2, w0, w1, w2):
        rows = [r0, r1, r2]
        gsem = [g0, g1, g2]
        wsem = [w0, w1, w2]

        wid = lax.axis_index("s") * NC + lax.axis_index("c")
        base = wid * TPW

        pltpu.sync_copy(xf_hbm.at[pl.ds(base, TPW)], idxbuf)

        def gather(s):
            return pltpu.async_copy(
                table_hbm.at[idxbuf.at[pl.ds(s * C, C)]], rows[s % 3], gsem[s % 3]
            )

        gdesc = {0: gather(0), 1: gather(1)}
        wdesc = {}

        for s in range(NSTEP):
            if s + 2 < NSTEP:
                if s - 1 >= 0:
                    wdesc[(s + 2) % 3].wait()
                gdesc[(s + 2) % 3] = gather(s + 2)
            gdesc[s % 3].wait()
            wdesc[s % 3] = pltpu.async_copy(
                rows[s % 3], out_hbm.at[pl.ds(base + s * C, C)], wsem[s % 3]
            )

        for s in range(NSTEP - 3, NSTEP):
            wdesc[s % 3].wait()

    return body(table, xf)


R = 512


def _tc_body(g_ref, x_ref, pe_ref, o_ref):
    s = jnp.where(x_ref[...] != 0, jnp.float32(SCALE), jnp.float32(0.0))
    o_ref[...] = g_ref[...] * s + pe_ref[...]


def _tc_fma(gathered, xcol, pe):
    return pl.pallas_call(
        _tc_body,
        out_shape=jax.ShapeDtypeStruct((B * L, D), jnp.float32),
        grid=(L // R, B),
        in_specs=[
            pl.BlockSpec((R, D), lambda i, b: (b * (L // R) + i, 0)),
            pl.BlockSpec((R, 1), lambda i, b: (b * (L // R) + i, 0)),
            pl.BlockSpec((R, D), lambda i, b: (i, 0)),
        ],
        out_specs=pl.BlockSpec((R, D), lambda i, b: (b * (L // R) + i, 0)),
    )(gathered, xcol, pe)


def _make_pe_rows():
    pos = np.arange(L, dtype=np.float32)[:, None].astype(np.float64)
    i = np.arange(D // 2, dtype=np.float32)[None, :].astype(np.float64)
    angle = (pos / np.power(10000.0, 2.0 * i / D)).astype(np.float32)
    pe = np.zeros((L, D), dtype=np.float32)
    pe[:, 0::2] = np.sin(angle, dtype=np.float32)
    pe[:, 1::2] = np.cos(angle, dtype=np.float32)
    return pe


_PE_ROWS = _make_pe_rows()


def kernel(x, embed_table):
    xf = x.reshape(B * L).astype(jnp.int32)
    gathered = _sc_gather(embed_table, xf)
    out = _tc_fma(gathered, xf.reshape(B * L, 1), _PE_ROWS)
    return out.reshape(B, L, D)
